# baseline (speedup 1.0000x reference)
import jax
import jax.numpy as jnp
from jax import lax
from jax.experimental import pallas as pl
from jax.experimental.pallas import tpu as pltpu
from jax.experimental.pallas import tpu_sc as plsc

NN = 10000
NE = 320000
PK = 16
MR = NE // PK
NR = NN // PK
BM = 2000
NC, NS = 2, 16
NW = NC * NS
EPW = NE // NW
C = 125
K = EPW // C
GSZ = 16
NG = K // GSZ
NPT = NN // NS

_f32 = jnp.float32


def _bd(w):
    return jnp.kron(jnp.eye(PK, dtype=w.dtype), w)


def _bt(b):
    return jnp.tile(b, PK)[None, :]


def _node_enc_body(x_ref, w1, b1, w2, b2, o_ref):
    z = jnp.dot(x_ref[...], w1[...], preferred_element_type=_f32) + b1[...]
    z = jnp.maximum(z, 0.0)
    h = jnp.dot(z, w2[...], preferred_element_type=_f32) + b2[...]
    o_ref[...] = jnp.maximum(h, 0.0)


def _edge_enc_body(ea_ref, w1, b1, w2, b2, o_ref):
    z = jnp.dot(ea_ref[...], w1[...], preferred_element_type=_f32) + b1[...]
    z = jnp.maximum(z, 0.0)
    z = jnp.dot(z, w2[...], preferred_element_type=_f32) + b2[...]
    o_ref[...] = jnp.maximum(z, 0.0)


def _rel_body(gd_ref, gs_ref, e_ref, w1d, w1s, w1e, b1, w2, b2, w3, b3, o_ref):
    z = (jnp.dot(gd_ref[...], w1d[...], preferred_element_type=_f32)
         + jnp.dot(gs_ref[...], w1s[...], preferred_element_type=_f32)
         + jnp.dot(e_ref[...], w1e[...], preferred_element_type=_f32)
         + b1[...])
    z = jnp.maximum(z, 0.0)
    z = jnp.maximum(jnp.dot(z, w2[...], preferred_element_type=_f32) + b2[...], 0.0)
    o_ref[...] = jnp.dot(z, w3[...], preferred_element_type=_f32) + b3[...]


def _obj_body(h_ref, p_ref, w1h, w1a, b1, w2, b2, w3, b3, o_ref):
    h = h_ref[...]
    p = p_ref[...]
    aggr = p[0 * NR:1 * NR]
    for k in range(1, 8):
        aggr = aggr + p[k * NR:(k + 1) * NR]
    z = (jnp.dot(h, w1h[...], preferred_element_type=_f32)
         + jnp.dot(aggr, w1a[...], preferred_element_type=_f32)
         + b1[...])
    z = jnp.maximum(z, 0.0)
    z = jnp.maximum(jnp.dot(z, w2[...], preferred_element_type=_f32) + b2[...], 0.0)
    delta = jnp.dot(z, w3[...], preferred_element_type=_f32) + b3[...]
    o_ref[...] = 0.5 * h + 0.5 * jnp.maximum(delta, 0.0)


def _final_body(e0, e1, e2, e3, e4, e5, q0, q1, q2, q3, q4, q5,
                b1, w2, b2, w3, b3, o_ref):
    es = (e0, e1, e2, e3, e4, e5)
    qs = (q0, q1, q2, q3, q4, q5)
    z = b1[...]
    for e, q in zip(es, qs):
        z = z + jnp.dot(e[...], q[...], preferred_element_type=_f32)
    z = jnp.maximum(z, 0.0)
    z = jnp.maximum(jnp.dot(z, w2[...], preferred_element_type=_f32) + b2[...], 0.0)
    z = jnp.dot(z, w3[...], preferred_element_type=_f32) + b3[...]
    o_ref[...] = 1.0 / (1.0 + jnp.exp(-z))


def _full(shape):
    return pl.BlockSpec(shape, lambda i: tuple(0 for _ in shape))


def _rows(width):
    return pl.BlockSpec((BM, width), lambda i: (i, 0))


NQ = 4
QH = EPW // NQ


def _sc_gather_body(h_hbm, dst4, src4, gd_hbm, gs_hbm, h_sh,
                    idxd, idxs, r0, r1, r2, r3, *sems):
    c = lax.axis_index("c")
    s = lax.axis_index("s")
    w = c * NS + s
    base = w * EPW
    nb = s * NPT
    rows = (r0, r1, r2, r3)
    sg = sems[0:4]
    sw = sems[4:8]
    sia, sib = sems[8], sems[9]

    cid = pltpu.async_copy(dst4.at[w], idxd, sia)
    cis = pltpu.async_copy(src4.at[w], idxs, sib)
    pltpu.sync_copy(h_hbm.at[pl.ds(nb, NPT)], h_sh.at[pl.ds(nb, NPT)])
    plsc.subcore_barrier()
    cid.wait()
    g = [pltpu.async_copy(h_sh.at[idxd.at[q]], rows[q], sg[q])
         for q in range(NQ)]
    cis.wait()
    wd = []
    for q in range(NQ):
        g[q].wait()
        wd.append(pltpu.async_copy(
            rows[q], gd_hbm.at[pl.ds(base + q * QH, QH)], sw[q]))
    for q in range(NQ):
        wd[q].wait()
        g[q] = pltpu.async_copy(h_sh.at[idxs.at[q]], rows[q], sg[q])
    for q in range(NQ):
        g[q].wait()
        wd[q] = pltpu.async_copy(
            rows[q], gs_hbm.at[pl.ds(base + q * QH, QH)], sw[q])
    for q in range(NQ):
        wd[q].wait()


def _sc_scatter_body(e_hbm, dst4, zeros_hbm, out_hbm, ag0, ag1, ag2, ag3,
                     idx_v, e0, e1, e2, e3, *sems):
    c = lax.axis_index("c")
    s = lax.axis_index("s")
    w = c * NS + s
    base = w * EPW
    nb = s * NPT
    ev = (e0, e1, e2, e3)
    banks = (ag0, ag1, ag2, ag3)
    se = sems[0:4]
    ss = sems[4:8]
    szs = sems[8:12]
    si = sems[12]

    cz = [pltpu.async_copy(zeros_hbm.at[pl.ds(nb, NPT)],
                           banks[q].at[pl.ds(nb, NPT)], szs[q])
          for q in range(NQ)]
    ci = pltpu.async_copy(dst4.at[w], idx_v, si)
    ce = [pltpu.async_copy(e_hbm.at[pl.ds(base + q * QH, QH)], ev[q], se[q])
          for q in range(NQ)]
    for z in cz:
        z.wait()
    plsc.subcore_barrier()
    ci.wait()
    sc = []
    for q in range(NQ):
        ce[q].wait()
        sc.append(pltpu.async_copy(
            ev[q], banks[q].at[idx_v.at[q]], ss[q], add=True))
    for q in range(NQ):
        sc[q].wait()

    plsc.subcore_barrier()
    for q in range(NQ):
        pltpu.sync_copy(banks[q].at[pl.ds(nb, NPT)],
                        out_hbm.at[c, q, pl.ds(nb, NPT)])


import functools


@functools.lru_cache(maxsize=None)
def _sc_kernels():
    mesh = plsc.VectorSubcoreMesh(
        core_axis_name="c", subcore_axis_name="s",
        num_cores=NC, num_subcores=NS)
    params = pltpu.CompilerParams(
        use_tc_tiling_on_sc=False, needs_layout_passes=False)
    gather = pl.kernel(
        _sc_gather_body,
        out_type=(jax.ShapeDtypeStruct((NE, 8), _f32),
                  jax.ShapeDtypeStruct((NE, 8), _f32)),
        mesh=mesh,
        compiler_params=params,
        scratch_types=(
            [pltpu.VMEM_SHARED((NN, 8), _f32)]
            + [pltpu.VMEM((NQ, QH), jnp.int32)] * 2
            + [pltpu.VMEM((QH, 8), _f32)] * 4
            + [pltpu.SemaphoreType.DMA] * 10
        ),
    )
    scatter = pl.kernel(
        _sc_scatter_body,
        out_type=jax.ShapeDtypeStruct((NC, NQ, NN, 8), _f32),
        mesh=mesh,
        compiler_params=params,
        scratch_types=(
            [pltpu.VMEM_SHARED((NN, 8), _f32)] * 4
            + [pltpu.VMEM((NQ, QH), jnp.int32)]
            + [pltpu.VMEM((QH, 8), _f32)] * 4
            + [pltpu.SemaphoreType.DMA] * 13
        ),
    )
    return gather, scatter


@jax.jit
def _run(x, edge_attr, edge_index, params):
    src3 = edge_index[0].astype(jnp.int32).reshape(NW, NQ, QH)
    dst3 = edge_index[1].astype(jnp.int32).reshape(NW, NQ, QH)

    (wn1, bn1), (wn2, bn2) = params['node_encoder']
    (we1, be1), (we2, be2) = params['edge_encoder']
    layers = []
    for lp in params['layers']:
        (rw1, rb1), (rw2, rb2), (rw3, rb3) = lp['relational']
        (ow1, ob1), (ow2, ob2), (ow3, ob3) = lp['object']
        layers.append(dict(
            w1d=_bd(rw1[0:8]), w1s=_bd(rw1[8:16]), w1e=_bd(rw1[16:24]),
            b1=_bt(rb1), w2=_bd(rw2), b2=_bt(rb2), w3=_bd(rw3), b3=_bt(rb3),
            ow1h=_bd(ow1[0:8]), ow1a=_bd(ow1[8:16]), ob1=_bt(ob1),
            ow2=_bd(ow2), ob2=_bt(ob2), ow3=_bd(ow3), ob3=_bt(ob3),
        ))
    (fw1, fb1), (fw2, fb2), (fw3, fb3) = params['W']
    fq = [_bd(fw1[8 * l:8 * (l + 1)]) for l in range(6)]

    h = pl.pallas_call(
        _node_enc_body,
        out_shape=jax.ShapeDtypeStruct((NN, 8), _f32),
    )(x, wn1, bn1[None, :], wn2, bn2[None, :])

    ea2 = edge_attr.reshape(MR, 256)
    e2 = pl.pallas_call(
        _edge_enc_body,
        grid=(MR // BM,),
        in_specs=[_rows(256), _full((256, 256)), _full((1, 256)),
                  _full((256, 128)), _full((1, 128))],
        out_specs=_rows(128),
        out_shape=jax.ShapeDtypeStruct((MR, 128), _f32),
    )(ea2, _bd(we1), _bt(be1), _bd(we2), _bt(be2))

    sc_gather, sc_scatter = _sc_kernels()
    zeros = jnp.zeros((NN, 8), _f32)
    e_list = []
    for l in range(6):
        lw = layers[l]
        gd, gs = sc_gather(h, dst3, src3)
        e2 = pl.pallas_call(
            _rel_body,
            grid=(MR // BM,),
            in_specs=[_rows(128), _rows(128), _rows(128),
                      _full((128, 256)), _full((128, 256)), _full((128, 256)),
                      _full((1, 256)), _full((256, 256)), _full((1, 256)),
                      _full((256, 128)), _full((1, 128))],
            out_specs=_rows(128),
            out_shape=jax.ShapeDtypeStruct((MR, 128), _f32),
        )(gd.reshape(MR, 128), gs.reshape(MR, 128), e2,
          lw['w1d'], lw['w1s'], lw['w1e'], lw['b1'],
          lw['w2'], lw['b2'], lw['w3'], lw['b3'])
        e_list.append(e2)
        p = sc_scatter(e2.reshape(NE, 8), dst3, zeros)
        h2 = pl.pallas_call(
            _obj_body,
            out_shape=jax.ShapeDtypeStruct((NR, 128), _f32),
        )(h.reshape(NR, 128), p.reshape(8 * NR, 128),
          lw['ow1h'], lw['ow1a'], lw['ob1'], lw['ow2'], lw['ob2'],
          lw['ow3'], lw['ob3'])
        h = h2.reshape(NN, 8)

    wout = pl.pallas_call(
        _final_body,
        grid=(MR // BM,),
        in_specs=([_rows(128)] * 6 + [_full((128, 256))] * 6
                  + [_full((1, 256)), _full((256, 256)), _full((1, 256)),
                     _full((256, 16)), _full((1, 16))]),
        out_specs=_rows(16),
        out_shape=jax.ShapeDtypeStruct((MR, 16), _f32),
    )(*e_list, *fq, _bt(fb1), _bd(fw2), _bt(fb2), _bd(fw3), _bt(fb3))
    return wout.reshape(NE)


def kernel(x, edge_attr, edge_index, params):
    return _run(x, edge_attr, edge_index, params)


# speedup vs baseline: 1.0215x; 1.0215x over previous
import jax
import jax.numpy as jnp
from jax import lax
from jax.experimental import pallas as pl
from jax.experimental.pallas import tpu as pltpu
from jax.experimental.pallas import tpu_sc as plsc

NN = 10000
NE = 320000
PK = 16
MR = NE // PK
NR = NN // PK
BM = 2000
NC, NS = 2, 16
NW = NC * NS
EPW = NE // NW
C = 125
K = EPW // C
GSZ = 16
NG = K // GSZ
NPT = NN // NS

_f32 = jnp.float32


def _bd(w):
    return jnp.kron(jnp.eye(PK, dtype=w.dtype), w)


def _bt(b):
    return jnp.tile(b, PK)[None, :]


def _node_enc_body(x_ref, w1, b1, w2, b2, o_ref):
    z = jnp.dot(x_ref[...], w1[...], preferred_element_type=_f32) + b1[...]
    z = jnp.maximum(z, 0.0)
    h = jnp.dot(z, w2[...], preferred_element_type=_f32) + b2[...]
    o_ref[...] = jnp.maximum(h, 0.0)


def _edge_enc_body(ea_ref, w1, b1, w2, b2, o_ref):
    z = jnp.dot(ea_ref[...], w1[...], preferred_element_type=_f32) + b1[...]
    z = jnp.maximum(z, 0.0)
    z = jnp.dot(z, w2[...], preferred_element_type=_f32) + b2[...]
    o_ref[...] = jnp.maximum(z, 0.0)


def _rel_body(gd_ref, gs_ref, e_ref, w1d, w1s, w1e, b1, w2, b2, w3, b3, o_ref):
    z = (jnp.dot(gd_ref[...], w1d[...], preferred_element_type=_f32)
         + jnp.dot(gs_ref[...], w1s[...], preferred_element_type=_f32)
         + jnp.dot(e_ref[...], w1e[...], preferred_element_type=_f32)
         + b1[...])
    z = jnp.maximum(z, 0.0)
    z = jnp.maximum(jnp.dot(z, w2[...], preferred_element_type=_f32) + b2[...], 0.0)
    o_ref[...] = jnp.dot(z, w3[...], preferred_element_type=_f32) + b3[...]


def _obj_body(h_ref, p_ref, w1h, w1a, b1, w2, b2, w3, b3, o_ref):
    h = h_ref[...]
    p = p_ref[...]
    aggr = p[0 * NR:1 * NR]
    for k in range(1, 4):
        aggr = aggr + p[k * NR:(k + 1) * NR]
    z = (jnp.dot(h, w1h[...], preferred_element_type=_f32)
         + jnp.dot(aggr, w1a[...], preferred_element_type=_f32)
         + b1[...])
    z = jnp.maximum(z, 0.0)
    z = jnp.maximum(jnp.dot(z, w2[...], preferred_element_type=_f32) + b2[...], 0.0)
    delta = jnp.dot(z, w3[...], preferred_element_type=_f32) + b3[...]
    o_ref[...] = 0.5 * h + 0.5 * jnp.maximum(delta, 0.0)


def _final_body(e0, e1, e2, e3, e4, e5, q0, q1, q2, q3, q4, q5,
                b1, w2, b2, w3, b3, o_ref):
    es = (e0, e1, e2, e3, e4, e5)
    qs = (q0, q1, q2, q3, q4, q5)
    z = b1[...]
    for e, q in zip(es, qs):
        z = z + jnp.dot(e[...], q[...], preferred_element_type=_f32)
    z = jnp.maximum(z, 0.0)
    z = jnp.maximum(jnp.dot(z, w2[...], preferred_element_type=_f32) + b2[...], 0.0)
    z = jnp.dot(z, w3[...], preferred_element_type=_f32) + b3[...]
    o_ref[...] = 1.0 / (1.0 + jnp.exp(-z))


def _full(shape):
    return pl.BlockSpec(shape, lambda i: tuple(0 for _ in shape))


def _rows(width):
    return pl.BlockSpec((BM, width), lambda i: (i, 0))


NQ = 4
QH = EPW // NQ


def _sc_gather_body(h_hbm, dst4, src4, gd_hbm, gs_hbm, h_sh,
                    idxd, idxs, r0, r1, r2, r3, *sems):
    c = lax.axis_index("c")
    s = lax.axis_index("s")
    w = c * NS + s
    base = w * EPW
    nb = s * NPT
    rows = (r0, r1, r2, r3)
    sg = sems[0:4]
    sw = sems[4:8]
    sia, sib = sems[8], sems[9]

    cid = pltpu.async_copy(dst4.at[w], idxd, sia)
    cis = pltpu.async_copy(src4.at[w], idxs, sib)
    pltpu.sync_copy(h_hbm.at[pl.ds(nb, NPT)], h_sh.at[pl.ds(nb, NPT)])
    plsc.subcore_barrier()
    cid.wait()
    g = [pltpu.async_copy(h_sh.at[idxd.at[q]], rows[q], sg[q])
         for q in range(NQ)]
    cis.wait()
    wd = []
    for q in range(NQ):
        g[q].wait()
        wd.append(pltpu.async_copy(
            rows[q], gd_hbm.at[pl.ds(base + q * QH, QH)], sw[q]))
    for q in range(NQ):
        wd[q].wait()
        g[q] = pltpu.async_copy(h_sh.at[idxs.at[q]], rows[q], sg[q])
    for q in range(NQ):
        g[q].wait()
        wd[q] = pltpu.async_copy(
            rows[q], gs_hbm.at[pl.ds(base + q * QH, QH)], sw[q])
    for q in range(NQ):
        wd[q].wait()


def _sc_scatter_body(e_hbm, dst4, zeros_hbm, out_hbm, ag0, ag1,
                     idx_v, e0, e1, e2, e3, *sems):
    c = lax.axis_index("c")
    s = lax.axis_index("s")
    w = c * NS + s
    base = w * EPW
    nb = s * NPT
    ev = (e0, e1, e2, e3)
    banks = (ag0, ag0, ag1, ag1)
    se = sems[0:4]
    ss = sems[4:8]
    szs = sems[8:12]
    si = sems[12]

    cz = [pltpu.async_copy(zeros_hbm.at[pl.ds(nb, NPT)],
                           banks[2 * q].at[pl.ds(nb, NPT)], szs[q])
          for q in range(2)]
    ci = pltpu.async_copy(dst4.at[w], idx_v, si)
    ce = [pltpu.async_copy(e_hbm.at[pl.ds(base + q * QH, QH)], ev[q], se[q])
          for q in range(NQ)]
    for z in cz:
        z.wait()
    plsc.subcore_barrier()
    ci.wait()
    sc = []
    for q in range(NQ):
        ce[q].wait()
        sc.append(pltpu.async_copy(
            ev[q], banks[q].at[idx_v.at[q]], ss[q], add=True))
    for q in range(NQ):
        sc[q].wait()

    plsc.subcore_barrier()
    for q in range(2):
        pltpu.sync_copy(banks[2 * q].at[pl.ds(nb, NPT)],
                        out_hbm.at[c, q, pl.ds(nb, NPT)])


import functools


@functools.lru_cache(maxsize=None)
def _sc_kernels():
    mesh = plsc.VectorSubcoreMesh(
        core_axis_name="c", subcore_axis_name="s",
        num_cores=NC, num_subcores=NS)
    params = pltpu.CompilerParams(
        use_tc_tiling_on_sc=False, needs_layout_passes=False)
    gather = pl.kernel(
        _sc_gather_body,
        out_type=(jax.ShapeDtypeStruct((NE, 8), _f32),
                  jax.ShapeDtypeStruct((NE, 8), _f32)),
        mesh=mesh,
        compiler_params=params,
        scratch_types=(
            [pltpu.VMEM_SHARED((NN, 8), _f32)]
            + [pltpu.VMEM((NQ, QH), jnp.int32)] * 2
            + [pltpu.VMEM((QH, 8), _f32)] * 4
            + [pltpu.SemaphoreType.DMA] * 10
        ),
    )
    scatter = pl.kernel(
        _sc_scatter_body,
        out_type=jax.ShapeDtypeStruct((NC, 2, NN, 8), _f32),
        mesh=mesh,
        compiler_params=params,
        scratch_types=(
            [pltpu.VMEM_SHARED((NN, 8), _f32)] * 2
            + [pltpu.VMEM((NQ, QH), jnp.int32)]
            + [pltpu.VMEM((QH, 8), _f32)] * 4
            + [pltpu.SemaphoreType.DMA] * 13
        ),
    )
    return gather, scatter


@jax.jit
def _run(x, edge_attr, edge_index, params):
    src3 = edge_index[0].astype(jnp.int32).reshape(NW, NQ, QH)
    dst3 = edge_index[1].astype(jnp.int32).reshape(NW, NQ, QH)

    (wn1, bn1), (wn2, bn2) = params['node_encoder']
    (we1, be1), (we2, be2) = params['edge_encoder']
    layers = []
    for lp in params['layers']:
        (rw1, rb1), (rw2, rb2), (rw3, rb3) = lp['relational']
        (ow1, ob1), (ow2, ob2), (ow3, ob3) = lp['object']
        layers.append(dict(
            w1d=_bd(rw1[0:8]), w1s=_bd(rw1[8:16]), w1e=_bd(rw1[16:24]),
            b1=_bt(rb1), w2=_bd(rw2), b2=_bt(rb2), w3=_bd(rw3), b3=_bt(rb3),
            ow1h=_bd(ow1[0:8]), ow1a=_bd(ow1[8:16]), ob1=_bt(ob1),
            ow2=_bd(ow2), ob2=_bt(ob2), ow3=_bd(ow3), ob3=_bt(ob3),
        ))
    (fw1, fb1), (fw2, fb2), (fw3, fb3) = params['W']
    fq = [_bd(fw1[8 * l:8 * (l + 1)]) for l in range(6)]

    h = pl.pallas_call(
        _node_enc_body,
        out_shape=jax.ShapeDtypeStruct((NN, 8), _f32),
    )(x, wn1, bn1[None, :], wn2, bn2[None, :])

    ea2 = edge_attr.reshape(MR, 256)
    e2 = pl.pallas_call(
        _edge_enc_body,
        grid=(MR // BM,),
        in_specs=[_rows(256), _full((256, 256)), _full((1, 256)),
                  _full((256, 128)), _full((1, 128))],
        out_specs=_rows(128),
        out_shape=jax.ShapeDtypeStruct((MR, 128), _f32),
    )(ea2, _bd(we1), _bt(be1), _bd(we2), _bt(be2))

    sc_gather, sc_scatter = _sc_kernels()
    zeros = jnp.zeros((NN, 8), _f32)
    e_list = []
    for l in range(6):
        lw = layers[l]
        gd, gs = sc_gather(h, dst3, src3)
        e2 = pl.pallas_call(
            _rel_body,
            grid=(MR // BM,),
            in_specs=[_rows(128), _rows(128), _rows(128),
                      _full((128, 256)), _full((128, 256)), _full((128, 256)),
                      _full((1, 256)), _full((256, 256)), _full((1, 256)),
                      _full((256, 128)), _full((1, 128))],
            out_specs=_rows(128),
            out_shape=jax.ShapeDtypeStruct((MR, 128), _f32),
        )(gd.reshape(MR, 128), gs.reshape(MR, 128), e2,
          lw['w1d'], lw['w1s'], lw['w1e'], lw['b1'],
          lw['w2'], lw['b2'], lw['w3'], lw['b3'])
        e_list.append(e2)
        p = sc_scatter(e2.reshape(NE, 8), dst3, zeros)
        h2 = pl.pallas_call(
            _obj_body,
            out_shape=jax.ShapeDtypeStruct((NR, 128), _f32),
        )(h.reshape(NR, 128), p.reshape(4 * NR, 128),
          lw['ow1h'], lw['ow1a'], lw['ob1'], lw['ow2'], lw['ob2'],
          lw['ow3'], lw['ob3'])
        h = h2.reshape(NN, 8)

    wout = pl.pallas_call(
        _final_body,
        grid=(MR // BM,),
        in_specs=([_rows(128)] * 6 + [_full((128, 256))] * 6
                  + [_full((1, 256)), _full((256, 256)), _full((1, 256)),
                     _full((256, 16)), _full((1, 16))]),
        out_specs=_rows(16),
        out_shape=jax.ShapeDtypeStruct((MR, 16), _f32),
    )(*e_list, *fq, _bt(fb1), _bd(fw2), _bt(fb2), _bd(fw3), _bt(fb3))
    return wout.reshape(NE)


def kernel(x, edge_attr, edge_index, params):
    return _run(x, edge_attr, edge_index, params)
